# baseline (speedup 1.0000x reference)
import functools

import jax
import jax.numpy as jnp
from jax import lax
from jax.experimental import pallas as pl
from jax.experimental.pallas import tpu as pltpu
from jax.experimental.pallas import tpu_sc as plsc

B, N, D, U = 128, 1024, 768, 1024
TILE_N = 512
NT = N // TILE_N


def _hvec_body(hid_ref, wh_ref, bias_ref, o_ref):
    o_ref[...] = (
        jnp.dot(hid_ref[...], wh_ref[...], preferred_element_type=jnp.float32)
        + bias_ref[...]
    )


def _hvec(hidden, Wh, bias2d):
    return pl.pallas_call(
        _hvec_body,
        out_shape=jax.ShapeDtypeStruct((B, U), jnp.float32),
    )(hidden, Wh, bias2d)


_M = B * NT


def _score_body(feat_ref, wf_ref, hb_ref, ws_ref, o_ref, acc_ref):
    i = pl.program_id(0)

    @pl.when(i > 0)
    def _epilogue():
        prev = acc_ref[1 - (i % 2)]
        t = jnp.tanh(prev + hb_ref[0])
        logit = jnp.dot(t, ws_ref[...],
                        preferred_element_type=jnp.float32)[:, 0]
        o_ref[0, 0, :] = logit

    @pl.when(i < _M)
    def _main():
        acc_ref[i % 2] = jnp.dot(feat_ref[0], wf_ref[...],
                                 preferred_element_type=jnp.float32)


def _scores(features4, Wf, hb, ws_pad):
    return pl.pallas_call(
        _score_body,
        grid=(_M + 1,),
        in_specs=[
            pl.BlockSpec((1, TILE_N, D), lambda i: (jnp.minimum(i, _M - 1), 0, 0)),
            pl.BlockSpec((D, U), lambda i: (0, 0)),
            pl.BlockSpec((1, 1, U), lambda i: (jnp.maximum(i - 1, 0) // NT, 0, 0)),
            pl.BlockSpec((U, 128), lambda i: (0, 0)),
        ],
        out_specs=pl.BlockSpec(
            (1, 1, TILE_N),
            lambda i: (jnp.maximum(i - 1, 0) // NT, 0, jnp.maximum(i - 1, 0) % NT),
        ),
        out_shape=jax.ShapeDtypeStruct((B, 1, N), jnp.float32),
        scratch_shapes=[pltpu.VMEM((2, TILE_N, U), jnp.float32)],
    )(features4, Wf, hb, ws_pad)


def _softmax_body(logits_ref, bs_ref, alpha_ref, idx_ref):
    x = logits_ref[...] + bs_ref[0]
    m = jnp.max(x, axis=1, keepdims=True)
    e = jnp.exp(x - m)
    s = jnp.sum(e, axis=1, keepdims=True)
    a = e / s
    alpha_ref[...] = a
    am = jnp.max(a, axis=1, keepdims=True)
    col = lax.broadcasted_iota(jnp.int32, (B, N), 1)
    loc = jnp.min(jnp.where(a == am, col, N), axis=1)
    row0 = lax.broadcasted_iota(jnp.int32, (1, B), 1) * N
    idx_ref[...] = row0 + loc[None, :]


def _softmax_argmax(logits, bs):
    return pl.pallas_call(
        _softmax_body,
        in_specs=[
            pl.BlockSpec((B, N), lambda: (0, 0)),
            pl.BlockSpec(memory_space=pltpu.SMEM),
        ],
        out_specs=[
            pl.BlockSpec((B, N), lambda: (0, 0)),
            pl.BlockSpec((1, B), lambda: (0, 0)),
        ],
        out_shape=[
            jax.ShapeDtypeStruct((B, N), jnp.float32),
            jax.ShapeDtypeStruct((1, B), jnp.int32),
        ],
    )(logits, bs)


_NWU = 16
_RPW = B // _NWU


def _make_gather():
    info = plsc.get_sparse_core_info()
    nc = info.num_cores
    mesh = plsc.VectorSubcoreMesh(core_axis_name="c", subcore_axis_name="s")

    @functools.partial(
        pl.kernel,
        mesh=mesh,
        out_type=jax.ShapeDtypeStruct((B, D), jnp.float32),
        scratch_types=[
            pltpu.VMEM((_RPW,), jnp.int32),
            pltpu.VMEM((_RPW, D), jnp.float32),
            pltpu.SemaphoreType.DMA,
        ],
    )
    def gather(table_hbm, idx_hbm, out_hbm, idx_v, rows_v, sem):
        wid = lax.axis_index("s") * nc + lax.axis_index("c")

        @pl.when(wid < _NWU)
        def _():
            base = wid * _RPW
            pltpu.sync_copy(idx_hbm.at[pl.ds(base, _RPW)], idx_v)
            pltpu.async_copy(table_hbm.at[idx_v], rows_v, sem).wait()
            pltpu.sync_copy(rows_v, out_hbm.at[pl.ds(base, _RPW)])

    return gather


_gather = _make_gather()


def kernel(features, hidden, Wf, bf, Wh, bh, Ws, bs):
    hvec = _hvec(hidden, Wh, bh.reshape(1, U)).reshape(B, 1, U)
    hb = hvec + bf.reshape(1, 1, U)
    ws_pad = jnp.zeros((U, 128), jnp.float32).at[:, 0].set(Ws[:, 0])
    features4 = features.reshape(_M, TILE_N, D)
    logits = _scores(features4, Wf, hb, ws_pad)
    alpha2d, idx2d = _softmax_argmax(logits.reshape(B, N), bs)
    table = features.reshape(B * N, D)
    context = _gather(table, idx2d.reshape(B))
    return (context, alpha2d.reshape(B, N, 1))


# speedup vs baseline: 1.1684x; 1.1684x over previous
import functools

import jax
import jax.numpy as jnp
from jax import lax
from jax.experimental import pallas as pl
from jax.experimental.pallas import tpu as pltpu
from jax.experimental.pallas import tpu_sc as plsc

B, N, D, U = 128, 1024, 768, 1024
TILE_N = 1024
NT = N // TILE_N


def _hvec_body(hid_ref, wh_ref, bias_ref, o_ref):
    o_ref[...] = (
        jnp.dot(hid_ref[...], wh_ref[...], preferred_element_type=jnp.float32)
        + bias_ref[...]
    )


def _hvec(hidden, Wh, bias2d):
    return pl.pallas_call(
        _hvec_body,
        out_shape=jax.ShapeDtypeStruct((B, U), jnp.float32),
    )(hidden, Wh, bias2d)


_M = B * NT


def _score_body(feat_ref, wf_ref, hb_ref, ws_ref, o_ref, acc_ref):
    i = pl.program_id(0)

    @pl.when(i > 0)
    def _epilogue():
        prev = acc_ref[1 - (i % 2)]
        t = jnp.tanh(prev + hb_ref[0])
        logit = jnp.dot(t, ws_ref[...],
                        preferred_element_type=jnp.float32)[:, 0]
        o_ref[0, 0, :] = logit

    @pl.when(i < _M)
    def _main():
        acc_ref[i % 2] = jnp.dot(feat_ref[0], wf_ref[...],
                                 preferred_element_type=jnp.float32)


def _scores(features4, Wf, hb, ws_pad):
    return pl.pallas_call(
        _score_body,
        grid=(_M + 1,),
        in_specs=[
            pl.BlockSpec((1, TILE_N, D), lambda i: (jnp.minimum(i, _M - 1), 0, 0)),
            pl.BlockSpec((D, U), lambda i: (0, 0)),
            pl.BlockSpec((1, 1, U), lambda i: (jnp.maximum(i - 1, 0) // NT, 0, 0)),
            pl.BlockSpec((U, 128), lambda i: (0, 0)),
        ],
        out_specs=pl.BlockSpec(
            (1, 1, TILE_N),
            lambda i: (jnp.maximum(i - 1, 0) // NT, 0, jnp.maximum(i - 1, 0) % NT),
        ),
        out_shape=jax.ShapeDtypeStruct((B, 1, N), jnp.float32),
        scratch_shapes=[pltpu.VMEM((2, TILE_N, U), jnp.float32)],
    )(features4, Wf, hb, ws_pad)


def _softmax_body(logits_ref, bs_ref, alpha_ref, idx_ref):
    x = logits_ref[...] + bs_ref[0]
    m = jnp.max(x, axis=1, keepdims=True)
    e = jnp.exp(x - m)
    s = jnp.sum(e, axis=1, keepdims=True)
    a = e / s
    alpha_ref[...] = a
    am = jnp.max(a, axis=1, keepdims=True)
    col = lax.broadcasted_iota(jnp.int32, (B, N), 1)
    loc = jnp.min(jnp.where(a == am, col, N), axis=1)
    row0 = lax.broadcasted_iota(jnp.int32, (1, B), 1) * N
    idx_ref[...] = row0 + loc[None, :]


def _softmax_argmax(logits, bs):
    return pl.pallas_call(
        _softmax_body,
        in_specs=[
            pl.BlockSpec((B, N), lambda: (0, 0)),
            pl.BlockSpec(memory_space=pltpu.SMEM),
        ],
        out_specs=[
            pl.BlockSpec((B, N), lambda: (0, 0)),
            pl.BlockSpec((1, B), lambda: (0, 0)),
        ],
        out_shape=[
            jax.ShapeDtypeStruct((B, N), jnp.float32),
            jax.ShapeDtypeStruct((1, B), jnp.int32),
        ],
    )(logits, bs)


_NWU = 16
_RPW = B // _NWU


def _make_gather():
    info = plsc.get_sparse_core_info()
    nc = info.num_cores
    mesh = plsc.VectorSubcoreMesh(core_axis_name="c", subcore_axis_name="s")

    @functools.partial(
        pl.kernel,
        mesh=mesh,
        out_type=jax.ShapeDtypeStruct((B, D), jnp.float32),
        scratch_types=[
            pltpu.VMEM((_RPW,), jnp.int32),
            pltpu.VMEM((_RPW, D), jnp.float32),
            pltpu.SemaphoreType.DMA,
        ],
    )
    def gather(table_hbm, idx_hbm, out_hbm, idx_v, rows_v, sem):
        wid = lax.axis_index("s") * nc + lax.axis_index("c")

        @pl.when(wid < _NWU)
        def _():
            base = wid * _RPW
            pltpu.sync_copy(idx_hbm.at[pl.ds(base, _RPW)], idx_v)
            pltpu.async_copy(table_hbm.at[idx_v], rows_v, sem).wait()
            pltpu.sync_copy(rows_v, out_hbm.at[pl.ds(base, _RPW)])

    return gather


_gather = _make_gather()


def kernel(features, hidden, Wf, bf, Wh, bh, Ws, bs):
    hvec = _hvec(hidden, Wh, bh.reshape(1, U)).reshape(B, 1, U)
    hb = hvec + bf.reshape(1, 1, U)
    ws_pad = jnp.zeros((U, 128), jnp.float32).at[:, 0].set(Ws[:, 0])
    features4 = features.reshape(_M, TILE_N, D)
    logits = _scores(features4, Wf, hb, ws_pad)
    alpha2d, idx2d = _softmax_argmax(logits.reshape(B, N), bs)
    table = features.reshape(B * N, D)
    context = _gather(table, idx2d.reshape(B))
    return (context, alpha2d.reshape(B, N, 1))


# speedup vs baseline: 1.2455x; 1.0659x over previous
import functools

import jax
import jax.numpy as jnp
from jax import lax
from jax.experimental import pallas as pl
from jax.experimental.pallas import tpu as pltpu
from jax.experimental.pallas import tpu_sc as plsc

B, N, D, U = 128, 1024, 768, 1024
TILE_N = 1024
NT = N // TILE_N


def _hvec_body(hid_ref, wh_ref, bias_ref, o_ref):
    o_ref[...] = (
        jnp.dot(hid_ref[...], wh_ref[...], preferred_element_type=jnp.float32)
        + bias_ref[...]
    )


def _hvec(hidden, Wh, bias2d):
    return pl.pallas_call(
        _hvec_body,
        out_shape=jax.ShapeDtypeStruct((B, U), jnp.float32),
    )(hidden, Wh, bias2d)


TB = 2
TM = TB * N
_M = B // TB


def _score_body(feat_ref, wf_ref, hb_ref, ws_ref, o_ref, acc_ref):
    i = pl.program_id(0)

    @pl.when(i > 0)
    def _epilogue():
        prev = acc_ref[1 - (i % 2)]
        t = jnp.tanh(prev.reshape(TB, N, U) + hb_ref[...])
        logit = jnp.dot(t.reshape(TM, U), ws_ref[...],
                        preferred_element_type=jnp.float32)[:, 0]
        o_ref[...] = logit.reshape(TB, 1, N)

    @pl.when(i < _M)
    def _main():
        acc_ref[i % 2] = jnp.dot(feat_ref[0], wf_ref[...],
                                 preferred_element_type=jnp.float32)


def _scores(features4, Wf, hb, ws_pad):
    return pl.pallas_call(
        _score_body,
        grid=(_M + 1,),
        in_specs=[
            pl.BlockSpec((1, TM, D), lambda i: (jnp.minimum(i, _M - 1), 0, 0)),
            pl.BlockSpec((D, U), lambda i: (0, 0)),
            pl.BlockSpec((TB, 1, U), lambda i: (jnp.maximum(i - 1, 0), 0, 0)),
            pl.BlockSpec((U, 128), lambda i: (0, 0)),
        ],
        out_specs=pl.BlockSpec(
            (TB, 1, N), lambda i: (jnp.maximum(i - 1, 0), 0, 0),
        ),
        out_shape=jax.ShapeDtypeStruct((B, 1, N), jnp.float32),
        scratch_shapes=[pltpu.VMEM((2, TM, U), jnp.float32)],
    )(features4, Wf, hb, ws_pad)


def _softmax_body(logits_ref, bs_ref, alpha_ref, idx_ref):
    x = logits_ref[...] + bs_ref[0]
    m = jnp.max(x, axis=1, keepdims=True)
    e = jnp.exp(x - m)
    s = jnp.sum(e, axis=1, keepdims=True)
    a = e / s
    alpha_ref[...] = a
    am = jnp.max(a, axis=1, keepdims=True)
    col = lax.broadcasted_iota(jnp.int32, (B, N), 1)
    loc = jnp.min(jnp.where(a == am, col, N), axis=1)
    row0 = lax.broadcasted_iota(jnp.int32, (1, B), 1) * N
    idx_ref[...] = row0 + loc[None, :]


def _softmax_argmax(logits, bs):
    return pl.pallas_call(
        _softmax_body,
        in_specs=[
            pl.BlockSpec((B, N), lambda: (0, 0)),
            pl.BlockSpec(memory_space=pltpu.SMEM),
        ],
        out_specs=[
            pl.BlockSpec((B, N), lambda: (0, 0)),
            pl.BlockSpec((1, B), lambda: (0, 0)),
        ],
        out_shape=[
            jax.ShapeDtypeStruct((B, N), jnp.float32),
            jax.ShapeDtypeStruct((1, B), jnp.int32),
        ],
    )(logits, bs)


_NWU = 16
_RPW = B // _NWU


def _make_gather():
    info = plsc.get_sparse_core_info()
    nc = info.num_cores
    mesh = plsc.VectorSubcoreMesh(core_axis_name="c", subcore_axis_name="s")

    @functools.partial(
        pl.kernel,
        mesh=mesh,
        out_type=jax.ShapeDtypeStruct((B, D), jnp.float32),
        scratch_types=[
            pltpu.VMEM((_RPW,), jnp.int32),
            pltpu.VMEM((_RPW, D), jnp.float32),
            pltpu.SemaphoreType.DMA,
        ],
    )
    def gather(table_hbm, idx_hbm, out_hbm, idx_v, rows_v, sem):
        wid = lax.axis_index("s") * nc + lax.axis_index("c")

        @pl.when(wid < _NWU)
        def _():
            base = wid * _RPW
            pltpu.sync_copy(idx_hbm.at[pl.ds(base, _RPW)], idx_v)
            pltpu.async_copy(table_hbm.at[idx_v], rows_v, sem).wait()
            pltpu.sync_copy(rows_v, out_hbm.at[pl.ds(base, _RPW)])

    return gather


_gather = _make_gather()


def kernel(features, hidden, Wf, bf, Wh, bh, Ws, bs):
    hvec = _hvec(hidden, Wh, bh.reshape(1, U)).reshape(B, 1, U)
    hb = hvec + bf.reshape(1, 1, U)
    ws_pad = jnp.zeros((U, 128), jnp.float32).at[:, 0].set(Ws[:, 0])
    features4 = features.reshape(_M, TM, D)
    logits = _scores(features4, Wf, hb, ws_pad)
    alpha2d, idx2d = _softmax_argmax(logits.reshape(B, N), bs)
    table = features.reshape(B * N, D)
    context = _gather(table, idx2d.reshape(B))
    return (context, alpha2d.reshape(B, N, 1))


# speedup vs baseline: 1.3340x; 1.0711x over previous
import functools

import jax
import jax.numpy as jnp
from jax import lax
from jax.experimental import pallas as pl
from jax.experimental.pallas import tpu as pltpu
from jax.experimental.pallas import tpu_sc as plsc

B, N, D, U = 128, 1024, 768, 1024
TILE_N = 1024
NT = N // TILE_N


def _hvec_body(hid_ref, wh_ref, bias_ref, o_ref):
    o_ref[...] = (
        jnp.dot(hid_ref[...], wh_ref[...], preferred_element_type=jnp.float32)
        + bias_ref[...]
    )


def _hvec(hidden, Wh, bias2d):
    return pl.pallas_call(
        _hvec_body,
        out_shape=jax.ShapeDtypeStruct((B, U), jnp.float32),
    )(hidden, Wh, bias2d)


TB = 2
TM = TB * N
_M = B // TB


def _score_body(feat_ref, wf_ref, hb_ref, ws_ref, bs_ref,
                alpha_ref, idx_ref, acc_ref, lgs_ref):
    i = pl.program_id(0)

    @pl.when(i > 0)
    def _epilogue():
        prev = acc_ref[1 - (i % 2)]
        t = jnp.tanh(prev.reshape(TB, N, U) + hb_ref[...])
        logit = jnp.dot(t.reshape(TM, U), ws_ref[...],
                        preferred_element_type=jnp.float32)[:, 0]
        lgs_ref[i - 1] = logit.reshape(TB, N)

    @pl.when(i < _M)
    def _main():
        acc_ref[i % 2] = jnp.dot(feat_ref[0], wf_ref[...],
                                 preferred_element_type=jnp.float32)

    @pl.when(i == _M)
    def _softmax():
        x = lgs_ref[...] + bs_ref[0]
        m = jnp.max(x, axis=2, keepdims=True)
        e = jnp.exp(x - m)
        s = jnp.sum(e, axis=2, keepdims=True)
        a = e / s
        alpha_ref[...] = a.reshape(B, 1, N)
        am = jnp.max(a, axis=2, keepdims=True)
        col = lax.broadcasted_iota(jnp.int32, (_M, TB, N), 2)
        loc = jnp.min(jnp.where(a == am, col, N), axis=2)
        row0 = lax.broadcasted_iota(jnp.int32, (1, B), 1) * N
        idx_ref[...] = row0 + loc.reshape(1, B)


def _scores(features4, Wf, hb, ws_pad, bs):
    return pl.pallas_call(
        _score_body,
        grid=(_M + 1,),
        in_specs=[
            pl.BlockSpec((1, TM, D), lambda i: (jnp.minimum(i, _M - 1), 0, 0)),
            pl.BlockSpec((D, U), lambda i: (0, 0)),
            pl.BlockSpec((TB, 1, U), lambda i: (jnp.maximum(i - 1, 0), 0, 0)),
            pl.BlockSpec((U, 128), lambda i: (0, 0)),
            pl.BlockSpec(memory_space=pltpu.SMEM),
        ],
        out_specs=[
            pl.BlockSpec((B, 1, N), lambda i: (0, 0, 0)),
            pl.BlockSpec((1, B), lambda i: (0, 0)),
        ],
        out_shape=[
            jax.ShapeDtypeStruct((B, 1, N), jnp.float32),
            jax.ShapeDtypeStruct((1, B), jnp.int32),
        ],
        scratch_shapes=[
            pltpu.VMEM((2, TM, U), jnp.float32),
            pltpu.VMEM((_M, TB, N), jnp.float32),
        ],
    )(features4, Wf, hb, ws_pad, bs)


def _softmax_body(logits_ref, bs_ref, alpha_ref, idx_ref):
    x = logits_ref[...] + bs_ref[0]
    m = jnp.max(x, axis=1, keepdims=True)
    e = jnp.exp(x - m)
    s = jnp.sum(e, axis=1, keepdims=True)
    a = e / s
    alpha_ref[...] = a
    am = jnp.max(a, axis=1, keepdims=True)
    col = lax.broadcasted_iota(jnp.int32, (B, N), 1)
    loc = jnp.min(jnp.where(a == am, col, N), axis=1)
    row0 = lax.broadcasted_iota(jnp.int32, (1, B), 1) * N
    idx_ref[...] = row0 + loc[None, :]


def _softmax_argmax(logits, bs):
    return pl.pallas_call(
        _softmax_body,
        in_specs=[
            pl.BlockSpec((B, N), lambda: (0, 0)),
            pl.BlockSpec(memory_space=pltpu.SMEM),
        ],
        out_specs=[
            pl.BlockSpec((B, N), lambda: (0, 0)),
            pl.BlockSpec((1, B), lambda: (0, 0)),
        ],
        out_shape=[
            jax.ShapeDtypeStruct((B, N), jnp.float32),
            jax.ShapeDtypeStruct((1, B), jnp.int32),
        ],
    )(logits, bs)


_NWU = 16
_RPW = B // _NWU


def _make_gather():
    info = plsc.get_sparse_core_info()
    nc = info.num_cores
    mesh = plsc.VectorSubcoreMesh(core_axis_name="c", subcore_axis_name="s")

    @functools.partial(
        pl.kernel,
        mesh=mesh,
        out_type=jax.ShapeDtypeStruct((B, D), jnp.float32),
        scratch_types=[
            pltpu.VMEM((_RPW,), jnp.int32),
            pltpu.VMEM((_RPW, D), jnp.float32),
            pltpu.SemaphoreType.DMA,
        ],
    )
    def gather(table_hbm, idx_hbm, out_hbm, idx_v, rows_v, sem):
        wid = lax.axis_index("s") * nc + lax.axis_index("c")

        @pl.when(wid < _NWU)
        def _():
            base = wid * _RPW
            pltpu.sync_copy(idx_hbm.at[pl.ds(base, _RPW)], idx_v)
            pltpu.async_copy(table_hbm.at[idx_v], rows_v, sem).wait()
            pltpu.sync_copy(rows_v, out_hbm.at[pl.ds(base, _RPW)])

    return gather


_gather = _make_gather()


def kernel(features, hidden, Wf, bf, Wh, bh, Ws, bs):
    hvec = _hvec(hidden, Wh, bh.reshape(1, U)).reshape(B, 1, U)
    hb = hvec + bf.reshape(1, 1, U)
    ws_pad = jnp.zeros((U, 128), jnp.float32).at[:, 0].set(Ws[:, 0])
    features4 = features.reshape(_M, TM, D)
    alpha3, idx2d = _scores(features4, Wf, hb, ws_pad, bs)
    table = features.reshape(B * N, D)
    context = _gather(table, idx2d.reshape(B))
    return (context, alpha3.reshape(B, N, 1))


# speedup vs baseline: 1.3433x; 1.0070x over previous
import functools

import jax
import jax.numpy as jnp
from jax import lax
from jax.experimental import pallas as pl
from jax.experimental.pallas import tpu as pltpu
from jax.experimental.pallas import tpu_sc as plsc

B, N, D, U = 128, 1024, 768, 1024
TILE_N = 1024
NT = N // TILE_N


def _hvec_body(hid_ref, wh_ref, bias_ref, o_ref):
    o_ref[...] = (
        jnp.dot(hid_ref[...], wh_ref[...], preferred_element_type=jnp.float32)
        + bias_ref[...]
    )


def _hvec(hidden, Wh, bias2d):
    return pl.pallas_call(
        _hvec_body,
        out_shape=jax.ShapeDtypeStruct((B, U), jnp.float32),
    )(hidden, Wh, bias2d)


TB = 2
TM = TB * N
_M = B // TB


def _score_body(feat_ref, wf_ref, hid_ref, wh_ref, hbias_ref, ws_ref, bs_ref,
                alpha_ref, idx_ref, acc_ref, lgs_ref, hb_ref):
    i = pl.program_id(0)

    @pl.when(i == 0)
    def _hidden_proj():
        hv = (jnp.dot(hid_ref[...], wh_ref[...],
                      preferred_element_type=jnp.float32)
              + hbias_ref[...])
        hb_ref[...] = hv.reshape(_M, TB, 1, U)

    @pl.when(i > 0)
    def _epilogue():
        prev = acc_ref[1 - (i % 2)]
        t = jnp.tanh(prev.reshape(TB, N, U) + hb_ref[i - 1])
        logit = jnp.dot(t.reshape(TM, U), ws_ref[...],
                        preferred_element_type=jnp.float32)[:, 0]
        lgs_ref[i - 1] = logit.reshape(TB, N)

    @pl.when(i < _M)
    def _main():
        acc_ref[i % 2] = jnp.dot(feat_ref[0], wf_ref[...],
                                 preferred_element_type=jnp.float32)

    @pl.when(i == _M)
    def _softmax():
        x = lgs_ref[...] + bs_ref[0]
        m = jnp.max(x, axis=2, keepdims=True)
        e = jnp.exp(x - m)
        s = jnp.sum(e, axis=2, keepdims=True)
        a = e / s
        alpha_ref[...] = a.reshape(B, 1, N)
        am = jnp.max(a, axis=2, keepdims=True)
        col = lax.broadcasted_iota(jnp.int32, (_M, TB, N), 2)
        loc = jnp.min(jnp.where(a == am, col, N), axis=2)
        row0 = lax.broadcasted_iota(jnp.int32, (1, B), 1) * N
        idx_ref[...] = row0 + loc.reshape(1, B)


def _scores(features4, Wf, hidden, Wh, hbias, ws_pad, bs):
    return pl.pallas_call(
        _score_body,
        grid=(_M + 1,),
        in_specs=[
            pl.BlockSpec((1, TM, D), lambda i: (jnp.minimum(i, _M - 1), 0, 0)),
            pl.BlockSpec((D, U), lambda i: (0, 0)),
            pl.BlockSpec((B, U), lambda i: (0, 0)),
            pl.BlockSpec((U, U), lambda i: (0, 0)),
            pl.BlockSpec((1, U), lambda i: (0, 0)),
            pl.BlockSpec((U, 128), lambda i: (0, 0)),
            pl.BlockSpec(memory_space=pltpu.SMEM),
        ],
        out_specs=[
            pl.BlockSpec((B, 1, N), lambda i: (0, 0, 0)),
            pl.BlockSpec((1, B), lambda i: (0, 0)),
        ],
        out_shape=[
            jax.ShapeDtypeStruct((B, 1, N), jnp.float32),
            jax.ShapeDtypeStruct((1, B), jnp.int32),
        ],
        scratch_shapes=[
            pltpu.VMEM((2, TM, U), jnp.float32),
            pltpu.VMEM((_M, TB, N), jnp.float32),
            pltpu.VMEM((_M, TB, 1, U), jnp.float32),
        ],
    )(features4, Wf, hidden, Wh, hbias, ws_pad, bs)


def _softmax_body(logits_ref, bs_ref, alpha_ref, idx_ref):
    x = logits_ref[...] + bs_ref[0]
    m = jnp.max(x, axis=1, keepdims=True)
    e = jnp.exp(x - m)
    s = jnp.sum(e, axis=1, keepdims=True)
    a = e / s
    alpha_ref[...] = a
    am = jnp.max(a, axis=1, keepdims=True)
    col = lax.broadcasted_iota(jnp.int32, (B, N), 1)
    loc = jnp.min(jnp.where(a == am, col, N), axis=1)
    row0 = lax.broadcasted_iota(jnp.int32, (1, B), 1) * N
    idx_ref[...] = row0 + loc[None, :]


def _softmax_argmax(logits, bs):
    return pl.pallas_call(
        _softmax_body,
        in_specs=[
            pl.BlockSpec((B, N), lambda: (0, 0)),
            pl.BlockSpec(memory_space=pltpu.SMEM),
        ],
        out_specs=[
            pl.BlockSpec((B, N), lambda: (0, 0)),
            pl.BlockSpec((1, B), lambda: (0, 0)),
        ],
        out_shape=[
            jax.ShapeDtypeStruct((B, N), jnp.float32),
            jax.ShapeDtypeStruct((1, B), jnp.int32),
        ],
    )(logits, bs)


_NWU = 16
_RPW = B // _NWU


def _make_gather():
    info = plsc.get_sparse_core_info()
    nc = info.num_cores
    mesh = plsc.VectorSubcoreMesh(core_axis_name="c", subcore_axis_name="s")

    @functools.partial(
        pl.kernel,
        mesh=mesh,
        out_type=jax.ShapeDtypeStruct((B, D), jnp.float32),
        scratch_types=[
            pltpu.VMEM((_RPW,), jnp.int32),
            pltpu.VMEM((_RPW, D), jnp.float32),
            pltpu.SemaphoreType.DMA,
        ],
    )
    def gather(table_hbm, idx_hbm, out_hbm, idx_v, rows_v, sem):
        wid = lax.axis_index("s") * nc + lax.axis_index("c")

        @pl.when(wid < _NWU)
        def _():
            base = wid * _RPW
            pltpu.sync_copy(idx_hbm.at[pl.ds(base, _RPW)], idx_v)
            pltpu.async_copy(table_hbm.at[idx_v], rows_v, sem).wait()
            pltpu.sync_copy(rows_v, out_hbm.at[pl.ds(base, _RPW)])

    return gather


_gather = _make_gather()


def kernel(features, hidden, Wf, bf, Wh, bh, Ws, bs):
    hbias = (bh + bf).reshape(1, U)
    ws_pad = jnp.zeros((U, 128), jnp.float32).at[:, 0].set(Ws[:, 0])
    features4 = features.reshape(_M, TM, D)
    alpha3, idx2d = _scores(features4, Wf, hidden, Wh, hbias, ws_pad, bs)
    table = features.reshape(B * N, D)
    context = _gather(table, idx2d.reshape(B))
    return (context, alpha3.reshape(B, N, 1))


# speedup vs baseline: 1.3492x; 1.0044x over previous
import functools

import jax
import jax.numpy as jnp
from jax import lax
from jax.experimental import pallas as pl
from jax.experimental.pallas import tpu as pltpu
from jax.experimental.pallas import tpu_sc as plsc

B, N, D, U = 128, 1024, 768, 1024


TB = 2
TM = TB * N
_M = B // TB


def _score_body(feat_ref, wf_ref, hid_ref, wh_ref, hbias_ref, ws_ref, bs_ref,
                alpha_ref, idx_ref, acc_ref, lgs_ref, hb_ref):
    i = pl.program_id(0)

    @pl.when(i == 0)
    def _hidden_proj():
        hv = (jnp.dot(hid_ref[...], wh_ref[...],
                      preferred_element_type=jnp.float32)
              + hbias_ref[...])
        hb_ref[...] = hv.reshape(_M, TB, 1, U)

    @pl.when(i > 0)
    def _epilogue():
        prev = acc_ref[1 - (i % 2)]
        t = jnp.tanh(prev.reshape(TB, N, U) + hb_ref[i - 1])
        logit = jnp.dot(t.reshape(TM, U), ws_ref[...],
                        preferred_element_type=jnp.float32)[:, 0]
        lgs_ref[i - 1] = logit.reshape(TB, N)

    @pl.when(i < _M)
    def _main():
        acc_ref[i % 2] = jnp.dot(feat_ref[0], wf_ref[...],
                                 preferred_element_type=jnp.float32)

    @pl.when(i == _M)
    def _softmax():
        x = lgs_ref[...] + bs_ref[0]
        m = jnp.max(x, axis=2, keepdims=True)
        e = jnp.exp(x - m)
        s = jnp.sum(e, axis=2, keepdims=True)
        a = e / s
        alpha_ref[...] = a.reshape(B, 1, N)
        am = jnp.max(a, axis=2, keepdims=True)
        col = lax.broadcasted_iota(jnp.int32, (_M, TB, N), 2)
        loc = jnp.min(jnp.where(a == am, col, N), axis=2)
        row0 = lax.broadcasted_iota(jnp.int32, (1, B), 1) * N
        idx_ref[...] = row0 + loc.reshape(1, B)


def _scores(features4, Wf, hidden, Wh, hbias, ws_pad, bs):
    return pl.pallas_call(
        _score_body,
        grid=(_M + 1,),
        in_specs=[
            pl.BlockSpec((1, TM, D), lambda i: (jnp.minimum(i, _M - 1), 0, 0)),
            pl.BlockSpec((D, U), lambda i: (0, 0)),
            pl.BlockSpec((B, U), lambda i: (0, 0)),
            pl.BlockSpec((U, U), lambda i: (0, 0)),
            pl.BlockSpec((1, U), lambda i: (0, 0)),
            pl.BlockSpec((U, 128), lambda i: (0, 0)),
            pl.BlockSpec(memory_space=pltpu.SMEM),
        ],
        out_specs=[
            pl.BlockSpec((B, 1, N), lambda i: (0, 0, 0)),
            pl.BlockSpec((1, B), lambda i: (0, 0)),
        ],
        out_shape=[
            jax.ShapeDtypeStruct((B, 1, N), jnp.float32),
            jax.ShapeDtypeStruct((1, B), jnp.int32),
        ],
        scratch_shapes=[
            pltpu.VMEM((2, TM, U), jnp.float32),
            pltpu.VMEM((_M, TB, N), jnp.float32),
            pltpu.VMEM((_M, TB, 1, U), jnp.float32),
        ],
    )(features4, Wf, hidden, Wh, hbias, ws_pad, bs)


_NWU = 16
_RPW = B // _NWU


def _make_gather():
    info = plsc.get_sparse_core_info()
    nc = info.num_cores
    mesh = plsc.VectorSubcoreMesh(core_axis_name="c", subcore_axis_name="s")

    @functools.partial(
        pl.kernel,
        mesh=mesh,
        out_type=jax.ShapeDtypeStruct((B, D), jnp.float32),
        scratch_types=[
            pltpu.VMEM((_RPW,), jnp.int32),
            pltpu.VMEM((_RPW, D), jnp.float32),
            pltpu.SemaphoreType.DMA,
        ],
    )
    def gather(table_hbm, idx_hbm, out_hbm, idx_v, rows_v, sem):
        wid = lax.axis_index("s") * nc + lax.axis_index("c")

        @pl.when(wid < _NWU)
        def _():
            base = wid * _RPW
            pltpu.sync_copy(idx_hbm.at[pl.ds(base, _RPW)], idx_v)
            pltpu.async_copy(table_hbm.at[idx_v], rows_v, sem).wait()
            pltpu.sync_copy(rows_v, out_hbm.at[pl.ds(base, _RPW)])

    return gather


_gather = _make_gather()


def kernel(features, hidden, Wf, bf, Wh, bh, Ws, bs):
    hbias = (bh + bf).reshape(1, U)
    ws_pad = jnp.zeros((U, 128), jnp.float32).at[:, 0].set(Ws[:, 0])
    features4 = features.reshape(_M, TM, D)
    alpha3, idx2d = _scores(features4, Wf, hidden, Wh, hbias, ws_pad, bs)
    table = features.reshape(B * N, D)
    context = _gather(table, idx2d.reshape(B))
    return (context, alpha3.reshape(B, N, 1))
